# baseline (speedup 1.0000x reference)
import functools

import jax
import jax.numpy as jnp
from jax.experimental import pallas as pl
from jax.experimental.pallas import tpu as pltpu

P = 0.1
SEED = 42


def _binary_entropy(p):
    p = jnp.asarray(p, dtype=jnp.float32)
    q = 1.0 - p
    min_real = jnp.finfo(jnp.float32).min
    log2_p = jnp.maximum(jnp.log2(p), min_real)
    log2_q = jnp.maximum(jnp.log2(q), min_real)
    return -p * log2_p - q * log2_q


def _erase_kernel(h_ref, msg_ref, mask_ref, ent_ref, out_ref, ent_out_ref):
    msg = msg_ref[...]
    m = mask_ref[...][:, :, None]
    col = jax.lax.broadcasted_iota(jnp.int32, msg.shape, 2)
    keep = 1.0 - m * (col >= 1).astype(jnp.float32)
    out_ref[:, :, :-1] = msg * keep
    p0 = msg[:, :, 0:1]
    out_ref[:, :, -1:] = m * (1.0 - p0)
    ent_out_ref[...] = ent_ref[...] + h_ref[0]


@jax.jit
def _run(messages, entropy, apply_noise):
    B, L, V = messages.shape
    noise_on = (jnp.asarray(apply_noise) != 0)
    target_mask = jax.random.uniform(jax.random.key(SEED), (B, L)) < P
    mask_f = (target_mask & noise_on).astype(jnp.float32)
    h = jnp.where(noise_on, _binary_entropy(P), 0.0).reshape(1)

    bm = 256
    grid = (B // bm,)
    out_shape = (
        jax.ShapeDtypeStruct((B, L, V + 1), messages.dtype),
        jax.ShapeDtypeStruct((B, L), entropy.dtype),
    )
    probs_out, ent_out = pl.pallas_call(
        _erase_kernel,
        grid_spec=pltpu.PrefetchScalarGridSpec(
            num_scalar_prefetch=1,
            grid=grid,
            in_specs=[
                pl.BlockSpec((bm, L, V), lambda i, h: (i, 0, 0)),
                pl.BlockSpec((bm, L), lambda i, h: (i, 0)),
                pl.BlockSpec((bm, L), lambda i, h: (i, 0)),
            ],
            out_specs=[
                pl.BlockSpec((bm, L, V + 1), lambda i, h: (i, 0, 0)),
                pl.BlockSpec((bm, L), lambda i, h: (i, 0)),
            ],
        ),
        out_shape=out_shape,
    )(h, messages, mask_f, entropy)
    return probs_out, ent_out


def kernel(messages, entropy, apply_noise):
    return _run(messages, entropy, apply_noise)
